# baseline (speedup 1.0000x reference)
import functools

import jax
import jax.numpy as jnp
from jax import lax
from jax.experimental import pallas as pl
from jax.experimental.pallas import tpu as pltpu
from jax.experimental.pallas import tpu_sc as plsc

NR = 512
N = 4096
D = 64
NC = 2
NS = 16
NW = NC * NS
ROWS_PER_W = N // NW
GROUPS = ROWS_PER_W // 16


def _neg_kernel(wo_ref, y_ref, rel_ref, out_ref, relm2_ref):
    ones_row = jnp.ones((1, D), jnp.float32)

    rel = rel_ref[...]
    rel_sq = jax.lax.dot_general(
        rel * rel, jnp.ones((D, 1), jnp.float32), (((1,), (0,)), ((), ())),
        precision=jax.lax.Precision.HIGHEST,
        preferred_element_type=jnp.float32,
    )
    rel_nrm = jnp.sqrt(rel_sq)
    rinv = 1.0 / jnp.maximum(rel_nrm, 1e-12)
    rel_n = rel * rinv
    relm2_ref[...] = -2.0 * rel_n
    rr = rel_nrm * rinv
    msq = rr * rr

    wo = wo_ref[...]
    wsq = jax.lax.dot_general(
        ones_row, wo * wo, (((1,), (1,)), ((), ())),
        preferred_element_type=jnp.float32,
    )
    wnrm = jnp.sqrt(wsq)
    inv = 1.0 / jnp.maximum(wnrm, 1e-12)
    r = wnrm * inv
    nsq = r * r

    st = jax.lax.dot_general(
        relm2_ref[...], wo, (((1,), (1,)), ((), ())),
        preferred_element_type=jnp.float32,
    )
    t = st * inv + msq

    y = y_ref[...]
    rows = jax.lax.broadcasted_iota(jnp.int32, t.shape, 0)
    is_pos = rows == y

    neg_t = jnp.min(jnp.where(is_pos, 1e9, t), axis=0, keepdims=True)
    neg_min = jnp.sqrt(jnp.maximum(nsq + neg_t, 0.0))
    out_ref[...] = jnp.sum(jnp.clip(1.0 - neg_min, 0.0, 9999.0)).reshape(1, 1)


def _nrsqrt(x):
    i = plsc.bitcast(x, jnp.int32)
    i = jnp.int32(0x5F3759DF) - lax.shift_right_arithmetic(i, 1)
    y = plsc.bitcast(i, jnp.float32)
    for _ in range(3):
        y = y * (1.5 - 0.5 * x * y * y)
    return y


def _pos_kernel(relf_hbm, y_hbm, wof_hbm, out_hbm, rel_v, idx_v, wo_v, acc_v):
    wid = lax.axis_index("s") * NC + lax.axis_index("c")
    base = wid * ROWS_PER_W

    pltpu.sync_copy(relf_hbm, rel_v)
    pltpu.sync_copy(y_hbm.at[pl.ds(base, ROWS_PER_W)], idx_v)
    pltpu.sync_copy(wof_hbm.at[pl.ds(base * D, ROWS_PER_W * D)], wo_v)

    iota = lax.iota(jnp.int32, 16)
    acc = jnp.zeros((16,), jnp.float32)
    for g in range(GROUPS):
        y_vec = idx_v[pl.ds(g * 16, 16)]
        ybase = y_vec * D
        wbase = (iota + g * 16) * D
        wsq = jnp.zeros((16,), jnp.float32)
        rsq = jnp.zeros((16,), jnp.float32)
        dot = jnp.zeros((16,), jnp.float32)
        for d in range(D):
            w = plsc.load_gather(wo_v, [wbase + d])
            c = plsc.load_gather(rel_v, [ybase + d])
            wsq = wsq + w * w
            rsq = rsq + c * c
            dot = dot + w * c
        inv_w = _nrsqrt(jnp.maximum(wsq, 1e-24))
        inv_r = _nrsqrt(jnp.maximum(rsq, 1e-24))
        nsq = wsq * inv_w * inv_w
        msq = rsq * inv_r * inv_r
        d2 = jnp.maximum(nsq + msq - 2.0 * dot * (inv_w * inv_r), 0.0)
        acc = acc + d2 * _nrsqrt(jnp.maximum(d2, 1e-30))
    acc_v[...] = acc
    pltpu.sync_copy(acc_v, out_hbm.at[wid])


_pos_call = pl.kernel(
    _pos_kernel,
    mesh=plsc.VectorSubcoreMesh(core_axis_name="c", subcore_axis_name="s"),
    out_type=jax.ShapeDtypeStruct((NW, 16), jnp.float32),
    compiler_params=pltpu.CompilerParams(needs_layout_passes=False),
    scratch_types=[
        pltpu.VMEM((NR * D,), jnp.float32),
        pltpu.VMEM((ROWS_PER_W,), jnp.int32),
        pltpu.VMEM((ROWS_PER_W * D,), jnp.float32),
        pltpu.VMEM((16,), jnp.float32),
    ],
)


@functools.partial(jax.jit, static_argnames=())
def kernel(wo, rel_weight, in_y):
    y_i32 = in_y.astype(jnp.int32)
    neg_sum = pl.pallas_call(
        _neg_kernel,
        out_shape=jax.ShapeDtypeStruct((1, 1), jnp.float32),
        scratch_shapes=[pltpu.VMEM((NR, D), jnp.float32)],
    )(wo, y_i32.reshape(1, N), rel_weight)
    pos_parts = _pos_call(rel_weight.reshape(NR * D), y_i32, wo.reshape(N * D))
    return (neg_sum[0, 0] + jnp.sum(pos_parts)) * (1.0 / N)


# speedup vs baseline: 3.8518x; 3.8518x over previous
import functools

import jax
import jax.numpy as jnp
from jax.experimental import pallas as pl
from jax.experimental.pallas import tpu as pltpu

NR = 512
N = 4096
D = 64


def _loss_kernel(wo_ref, y_ref, rel_ref, out_ref, relm2_ref, msq_ref):
    ones_row = jnp.ones((1, D), jnp.float32)

    rel = rel_ref[...]
    rel_sq = jax.lax.dot_general(
        rel * rel, jnp.ones((D, 1), jnp.float32), (((1,), (0,)), ((), ())),
        precision=jax.lax.Precision.HIGHEST,
        preferred_element_type=jnp.float32,
    )
    rel_nrm = jnp.sqrt(rel_sq)
    rinv = 1.0 / jnp.maximum(rel_nrm, 1e-12)
    rel_n = rel * rinv
    relm2_ref[...] = -2.0 * rel_n
    rr = rel_nrm * rinv
    msq_ref[...] = rr * rr

    wo = wo_ref[...]
    wsq = jax.lax.dot_general(
        ones_row, wo * wo, (((1,), (1,)), ((), ())),
        preferred_element_type=jnp.float32,
    )
    wnrm = jnp.sqrt(wsq)
    inv = 1.0 / jnp.maximum(wnrm, 1e-12)
    r = wnrm * inv
    nsq = r * r

    st = jax.lax.dot_general(
        relm2_ref[...], wo, (((1,), (1,)), ((), ())),
        preferred_element_type=jnp.float32,
    )
    t = st * inv + msq_ref[...]

    y = y_ref[...]
    rows = jax.lax.broadcasted_iota(jnp.int32, t.shape, 0)
    is_pos = rows == y

    neg_t = jnp.min(jnp.where(is_pos, 1e9, t), axis=0, keepdims=True)
    pos_t = jnp.sum(jnp.where(is_pos, t, 0.0), axis=0, keepdims=True)

    neg_min = jnp.sqrt(jnp.maximum(nsq + neg_t, 0.0))
    pos_d = jnp.sqrt(jnp.maximum(nsq + pos_t, 0.0))

    per_row = pos_d + jnp.clip(1.0 - neg_min, 0.0, 9999.0)
    out_ref[...] = jnp.sum(per_row).reshape(1, 1) * (1.0 / N)


@functools.partial(jax.jit, static_argnames=())
def kernel(wo, rel_weight, in_y):
    y2 = in_y.astype(jnp.int32).reshape(1, N)
    out = pl.pallas_call(
        _loss_kernel,
        out_shape=jax.ShapeDtypeStruct((1, 1), jnp.float32),
        scratch_shapes=[
            pltpu.VMEM((NR, D), jnp.float32),
            pltpu.VMEM((NR, 1), jnp.float32),
        ],
    )(wo, y2, rel_weight)
    return out[0, 0]


# speedup vs baseline: 4.6345x; 1.2032x over previous
import functools

import jax
import jax.numpy as jnp
from jax.experimental import pallas as pl
from jax.experimental.pallas import tpu as pltpu

NR = 512
N = 4096
D = 64


def _loss_kernel(wo_ref, y_ref, rel_ref, out_ref, relm2_ref, msq_ref):
    ones_row = jnp.ones((1, D), jnp.bfloat16)

    rel = rel_ref[...]
    rel_sq = jax.lax.dot_general(
        rel * rel, jnp.ones((D, 1), jnp.float32), (((1,), (0,)), ((), ())),
        precision=jax.lax.Precision.HIGHEST,
        preferred_element_type=jnp.float32,
    )
    rel_nrm = jnp.sqrt(rel_sq)
    rinv = 1.0 / jnp.maximum(rel_nrm, 1e-12)
    rel_n = rel * rinv
    relm2_ref[...] = (-2.0 * rel_n).astype(jnp.bfloat16)
    rr = rel_nrm * rinv
    msq_ref[...] = rr * rr

    wo = wo_ref[...]
    wsq = jax.lax.dot_general(
        ones_row, wo * wo, (((1,), (1,)), ((), ())),
        preferred_element_type=jnp.float32,
    )
    wnrm = jnp.sqrt(wsq)
    inv = 1.0 / jnp.maximum(wnrm, 1e-12)
    r = wnrm * inv
    nsq = r * r

    st = jax.lax.dot_general(
        relm2_ref[...], wo, (((1,), (1,)), ((), ())),
        preferred_element_type=jnp.float32,
    )
    t = st * inv + msq_ref[...]

    y = y_ref[...]
    rows = jax.lax.broadcasted_iota(jnp.int32, t.shape, 0)
    is_pos = rows == y

    neg_t = jnp.min(jnp.where(is_pos, 1e9, t), axis=0, keepdims=True)
    pos_t = jnp.sum(jnp.where(is_pos, t, 0.0), axis=0, keepdims=True)

    neg_min = jnp.sqrt(jnp.maximum(nsq + neg_t, 0.0))
    pos_d = jnp.sqrt(jnp.maximum(nsq + pos_t, 0.0))

    per_row = pos_d + jnp.clip(1.0 - neg_min, 0.0, 9999.0)
    out_ref[...] = jnp.sum(per_row).reshape(1, 1) * (1.0 / N)


@functools.partial(jax.jit, static_argnames=())
def kernel(wo, rel_weight, in_y):
    y2 = in_y.astype(jnp.int32).reshape(1, N)
    wo = wo.astype(jnp.bfloat16)
    out = pl.pallas_call(
        _loss_kernel,
        out_shape=jax.ShapeDtypeStruct((1, 1), jnp.float32),
        scratch_shapes=[
            pltpu.VMEM((NR, D), jnp.bfloat16),
            pltpu.VMEM((NR, 1), jnp.float32),
        ],
    )(wo, y2, rel_weight)
    return out[0, 0]
